# Initial kernel scaffold; baseline (speedup 1.0000x reference)
import jax
import jax.numpy as jnp
from jax.experimental import pallas as pl


def kernel(x_drug, dd_edge_index, dd_edge_type, dd_range_list, d_norm, x_prot, pp_edge_index, dp_edge_index, dp_range_list, gcn_w1, gcn_b1, gcn_w2, gcn_b2, embed, hgcn_w, basis1, att1, root1, basis2, att2, root2):
    raise NotImplementedError("write your pallas kernel here")


# speedup vs baseline: 15.1764x; 15.1764x over previous
import functools

import jax
import jax.numpy as jnp
from jax import lax
from jax.experimental import pallas as pl
from jax.experimental.pallas import tpu as pltpu
from jax.experimental.pallas import tpu_sc as plsc

N_PROT = 10000
N_DRUG = 10000
NUM_ET = 64

NC = 2
NS = 16
NW = NC * NS
SUB = 128
G = 8
EDGE_QUANTUM = NW * SUB * G

M_PAD = 10240
SINK = M_PAD - 1


def _pad_edges(e):
    return -(-e // EDGE_QUANTUM) * EDGE_QUANTUM


@functools.cache
def _sc_gather_scatter(d, e_pad, n_out):
    nsub_w = e_pad // (NW * SUB)
    ngroups = nsub_w // G
    rpt = n_out // NS
    mesh = plsc.VectorSubcoreMesh(core_axis_name="c", subcore_axis_name="s")

    @functools.partial(
        pl.kernel,
        out_type=jax.ShapeDtypeStruct((NC, n_out, d), jnp.float32),
        mesh=mesh,
        scratch_types=[
            pltpu.VMEM((G, SUB), jnp.int32),
            pltpu.VMEM((G, SUB), jnp.int32),
            pltpu.VMEM((G * SUB, d), jnp.float32),
            pltpu.VMEM((rpt, d), jnp.float32),
            pltpu.VMEM_SHARED((n_out, d), jnp.float32),
            pltpu.SemaphoreType.DMA,
            pltpu.SemaphoreType.DMA,
            pltpu.SemaphoreType.DMA,
        ],
        compiler_params=pltpu.CompilerParams(use_tc_tiling_on_sc=False),
    )
    def k(table, gidx, didx, zeros, out, gbuf, dbuf, rows, obuf, accum,
          isem, gsem, ssem):
        cid = lax.axis_index("c")
        sid = lax.axis_index("s")
        wid = sid * NC + cid
        pltpu.sync_copy(zeros.at[pl.ds(sid * rpt, rpt)], obuf)
        pltpu.sync_copy(obuf, accum.at[pl.ds(sid * rpt, rpt)])
        plsc.subcore_barrier()

        def body(g, carry):
            e0 = (wid * nsub_w + g * G) * SUB
            cps = []
            for j in range(G):
                cps.append(pltpu.async_copy(
                    gidx.at[pl.ds(e0 + j * SUB, SUB)], gbuf.at[j], isem))
                cps.append(pltpu.async_copy(
                    didx.at[pl.ds(e0 + j * SUB, SUB)], dbuf.at[j], isem))
            for c in cps:
                c.wait()
            cps = [pltpu.async_copy(table.at[gbuf.at[j]],
                                    rows.at[pl.ds(j * SUB, SUB)], gsem)
                   for j in range(G)]
            for c in cps:
                c.wait()
            cps = [pltpu.async_copy(rows.at[pl.ds(j * SUB, SUB)],
                                    accum.at[dbuf.at[j]], ssem, add=True)
                   for j in range(G)]
            for c in cps:
                c.wait()
            return carry

        lax.fori_loop(0, ngroups, body, 0)
        plsc.subcore_barrier()
        pltpu.sync_copy(accum.at[pl.ds(sid * rpt, rpt)], obuf)
        pltpu.sync_copy(obuf, out.at[cid, pl.ds(sid * rpt, rpt)])

    return k


@functools.cache
def _sc_counts(e_pad, n_out):
    d = 16
    nsub_w = e_pad // (NW * SUB)
    ngroups = nsub_w // G
    rpt = n_out // NS
    mesh = plsc.VectorSubcoreMesh(core_axis_name="c", subcore_axis_name="s")

    @functools.partial(
        pl.kernel,
        out_type=jax.ShapeDtypeStruct((NC, n_out, d), jnp.float32),
        mesh=mesh,
        scratch_types=[
            pltpu.VMEM((G, SUB), jnp.int32),
            pltpu.VMEM((SUB, d), jnp.float32),
            pltpu.VMEM((rpt, d), jnp.float32),
            pltpu.VMEM_SHARED((n_out, d), jnp.float32),
            pltpu.SemaphoreType.DMA,
            pltpu.SemaphoreType.DMA,
        ],
        compiler_params=pltpu.CompilerParams(use_tc_tiling_on_sc=False),
    )
    def k(ones, didx, zeros, out, dbuf, obuf_ones, obuf, accum, isem, ssem):
        cid = lax.axis_index("c")
        sid = lax.axis_index("s")
        wid = sid * NC + cid
        pltpu.sync_copy(ones, obuf_ones)
        pltpu.sync_copy(zeros.at[pl.ds(sid * rpt, rpt)], obuf)
        pltpu.sync_copy(obuf, accum.at[pl.ds(sid * rpt, rpt)])
        plsc.subcore_barrier()

        def body(g, carry):
            e0 = (wid * nsub_w + g * G) * SUB
            cps = [pltpu.async_copy(
                didx.at[pl.ds(e0 + j * SUB, SUB)], dbuf.at[j], isem)
                for j in range(G)]
            for c in cps:
                c.wait()
            cps = [pltpu.async_copy(obuf_ones, accum.at[dbuf.at[j]], ssem,
                                    add=True)
                   for j in range(G)]
            for c in cps:
                c.wait()
            return carry

        lax.fori_loop(0, ngroups, body, 0)
        plsc.subcore_barrier()
        pltpu.sync_copy(accum.at[pl.ds(sid * rpt, rpt)], obuf)
        pltpu.sync_copy(obuf, out.at[cid, pl.ds(sid * rpt, rpt)])

    return k


def _pad1(a, n, fill):
    return jnp.concatenate(
        [a, jnp.full((n - a.shape[0],), fill, a.dtype)])


def kernel(x_drug, dd_edge_index, dd_edge_type, dd_range_list, d_norm,
           x_prot, pp_edge_index, dp_edge_index, dp_range_list,
           gcn_w1, gcn_b1, gcn_w2, gcn_b2, embed, hgcn_w,
           basis1, att1, root1, basis2, att2, root2):
    e_pp = pp_edge_index.shape[1]
    e_dd = dd_edge_index.shape[1]
    e_dp = dp_edge_index.shape[1]

    pp_src, pp_dst = pp_edge_index[0], pp_edge_index[1]
    dd_src, dd_dst = dd_edge_index[0], dd_edge_index[1]
    dp_src, dp_dst = dp_edge_index[0], dp_edge_index[1]

    e_pp_pad = _pad_edges(e_pp)
    e_dd_pad = _pad_edges(e_dd)
    e_dp_pad = _pad_edges(e_dp)
    gidx_pp = _pad1(pp_src, e_pp_pad, 0)
    didx_pp = _pad1(pp_dst, e_pp_pad, SINK)
    gidx_dd = _pad1(dd_src * NUM_ET + dd_edge_type, e_dd_pad, 0)
    didx_dd = _pad1(dd_dst, e_dd_pad, SINK)
    gidx_dp = _pad1(dp_src, e_dp_pad, 0)
    didx_dp = _pad1(dp_dst - N_PROT, e_dp_pad, SINK)

    e_cnt = e_pp + e_dd + e_dp
    e_cnt_pad = _pad_edges(e_cnt)
    didx_cnt = _pad1(
        jnp.concatenate([pp_dst, dd_dst + M_PAD,
                         dp_dst + (2 * M_PAD - N_PROT)]),
        e_cnt_pad, 3 * M_PAD - 1)

    zeros_c = jnp.zeros((3 * M_PAD, 16), jnp.float32)
    zeros_32 = jnp.zeros((M_PAD, 32), jnp.float32)
    zeros_16 = jnp.zeros((M_PAD, 16), jnp.float32)
    ones_r = jnp.ones((SUB, 16), jnp.float32)

    cp = _sc_counts(e_cnt_pad, 3 * M_PAD)(ones_r, didx_cnt, zeros_c)
    cnt = cp[0, :, 0] + cp[1, :, 0]
    cnt_pp = cnt[:N_PROT]
    cnt_dd = jnp.maximum(cnt[M_PAD:M_PAD + N_DRUG], 1.0)
    cnt_dp = jnp.maximum(cnt[2 * M_PAD:2 * M_PAD + N_DRUG], 1.0)
    dinv = lax.rsqrt(cnt_pp + 1.0)

    gs32_pp = _sc_gather_scatter(32, e_pp_pad, M_PAD)
    gs16_pp = _sc_gather_scatter(16, e_pp_pad, M_PAD)
    gs32_dd = _sc_gather_scatter(32, e_dd_pad, M_PAD)
    gs16_dd = _sc_gather_scatter(16, e_dd_pad, M_PAD)
    gs16_dp = _sc_gather_scatter(16, e_dp_pad, M_PAD)

    g1 = (x_prot @ gcn_w1) * dinv[:, None]
    s1p = gs32_pp(g1, gidx_pp, didx_pp, zeros_32)
    s1 = (s1p[0] + s1p[1])[:N_PROT]
    xp1 = jax.nn.relu(dinv[:, None] * (s1 + g1) + gcn_b1)

    g2 = (xp1 @ gcn_w2) * dinv[:, None]
    s2p = gs16_pp(g2, gidx_pp, didx_pp, zeros_16)
    s2 = (s2p[0] + s2p[1])[:N_PROT]
    xp2 = dinv[:, None] * (s2 + g2) + gcn_b2

    s3p = gs16_dp(xp2, gidx_dp, didx_dp, zeros_16)
    s3 = (s3p[0] + s3p[1])[:N_DRUG]
    x_hier = (s3 / cnt_dp[:, None]) @ hgcn_w

    xd0 = (x_drug @ embed) / d_norm[:, None]
    z1 = jnp.concatenate([xd0, x_hier], axis=1)

    nb = basis1.shape[0]
    w1 = (att1 @ basis1.reshape(nb, -1)).reshape(NUM_ET, z1.shape[1], -1)
    w1cat = jnp.transpose(w1, (1, 0, 2)).reshape(z1.shape[1], -1)
    y1 = (z1 @ w1cat).reshape(N_DRUG * NUM_ET, -1)
    s4p = gs32_dd(y1, gidx_dd, didx_dd, zeros_32)
    s4 = (s4p[0] + s4p[1])[:N_DRUG]
    xd1 = jax.nn.relu(s4 / cnt_dd[:, None] + z1 @ root1)

    w2 = (att2 @ basis2.reshape(nb, -1)).reshape(NUM_ET, xd1.shape[1], -1)
    w2cat = jnp.transpose(w2, (1, 0, 2)).reshape(xd1.shape[1], -1)
    y2 = (xd1 @ w2cat).reshape(N_DRUG * NUM_ET, -1)
    s5p = gs16_dd(y2, gidx_dd, didx_dd, zeros_16)
    s5 = (s5p[0] + s5p[1])[:N_DRUG]
    return s5 / cnt_dd[:, None] + xd1 @ root2
